# baseline (speedup 1.0000x reference)
import functools

import jax
import jax.numpy as jnp
from jax.experimental import pallas as pl

_AA = 21
_FC = 256
_MT = 1000
_KT = 128


def _pad2(a, rows, cols):
    r, c = a.shape
    if r == rows and c == cols:
        return a
    return jnp.pad(a, ((0, rows - r), (0, cols - c)))


def _pad_cols(a, cols):
    return _pad2(a, a.shape[0], cols)


def _mm_kernel(x_ref, w_ref, b_ref, o_ref, *, nk, relu):
    k = pl.program_id(1)

    @pl.when(k == 0)
    def _():
        o_ref[...] = jnp.zeros_like(o_ref)

    o_ref[...] += jnp.dot(x_ref[...], w_ref[...],
                          preferred_element_type=jnp.float32)

    @pl.when(k == nk - 1)
    def _():
        r = o_ref[...] + b_ref[...]
        if relu:
            r = jnp.maximum(r, 0.0)
        o_ref[...] = r


def _mm(x, w, b, relu):
    m, kdim = x.shape
    n = w.shape[1]
    nm = m // _MT
    kt = _KT if kdim % _KT == 0 else kdim
    nk = kdim // kt
    return pl.pallas_call(
        functools.partial(_mm_kernel, nk=nk, relu=relu),
        grid=(nm, nk),
        in_specs=[
            pl.BlockSpec((_MT, kt), lambda i, k: (i, k)),
            pl.BlockSpec((kt, n), lambda i, k: (k, 0)),
            pl.BlockSpec((1, n), lambda i, k: (0, 0)),
        ],
        out_specs=pl.BlockSpec((_MT, n), lambda i, k: (i, 0)),
        out_shape=jax.ShapeDtypeStruct((m, n), jnp.float32),
    )(x, w, b.reshape(1, -1))


_EB = 5000


def _deg_kernel(dst_ref, o_ref, *, ne):
    eb = pl.program_id(0)

    @pl.when(eb == 0)
    def _():
        o_ref[...] = jnp.zeros_like(o_ref)

    def body(e, carry):
        d = dst_ref[pl.ds(e, 1), :][0, 0]
        o_ref[pl.ds(d, 1), :] = o_ref[pl.ds(d, 1), :] + 1.0
        return carry

    jax.lax.fori_loop(0, dst_ref.shape[0], body, 0)

    @pl.when(eb == ne - 1)
    def _():
        o_ref[...] = jax.lax.rsqrt(jnp.maximum(o_ref[...], 1e-12))


def _deg(dst, n):
    e = dst.shape[0]
    ne = e // _EB
    return pl.pallas_call(
        functools.partial(_deg_kernel, ne=ne),
        grid=(ne,),
        in_specs=[pl.BlockSpec((_EB, 1), lambda i: (i, 0))],
        out_specs=pl.BlockSpec((n, 1), lambda i: (0, 0)),
        out_shape=jax.ShapeDtypeStruct((n, 1), jnp.float32),
    )(dst)


def _agg_kernel(src_ref, dst_ref, dinv_ref, hw_ref, b_ref, o_ref, *, ne):
    eb = pl.program_id(1)

    @pl.when(eb == 0)
    def _():
        o_ref[...] = jnp.zeros_like(o_ref)

    def body(e, carry):
        s = src_ref[pl.ds(e, 1), :][0, 0]
        d = dst_ref[pl.ds(e, 1), :][0, 0]
        w = dinv_ref[pl.ds(s, 1), :][0, 0] * dinv_ref[pl.ds(d, 1), :][0, 0]
        row = hw_ref[pl.ds(s, 1), :]
        o_ref[pl.ds(d, 1), :] = o_ref[pl.ds(d, 1), :] + row * w
        return carry

    jax.lax.fori_loop(0, src_ref.shape[0], body, 0)

    @pl.when(eb == ne - 1)
    def _():
        o_ref[...] = jnp.maximum(o_ref[...] + b_ref[...], 0.0)


def _aggregate(hw, src, dst, dinv, b):
    n, f = hw.shape
    e = src.shape[0]
    nf = f // _FC
    ne = e // _EB
    return pl.pallas_call(
        functools.partial(_agg_kernel, ne=ne),
        grid=(nf, ne),
        in_specs=[
            pl.BlockSpec((_EB, 1), lambda j, i: (i, 0)),
            pl.BlockSpec((_EB, 1), lambda j, i: (i, 0)),
            pl.BlockSpec((n, 1), lambda j, i: (0, 0)),
            pl.BlockSpec((n, _FC), lambda j, i: (0, j)),
            pl.BlockSpec((1, _FC), lambda j, i: (0, j)),
        ],
        out_specs=pl.BlockSpec((n, _FC), lambda j, i: (0, j)),
        out_shape=jax.ShapeDtypeStruct((n, f), jnp.float32),
    )(src, dst, dinv, hw, b.reshape(1, -1))


def _gcn_conv(h, src, dst, dinv, w_pad, b_pad):
    hw = _mm(h, w_pad, jnp.zeros((w_pad.shape[1],), jnp.float32), relu=False)
    return _aggregate(hw, src, dst, dinv, b_pad)


def _pool_kernel(batch_ref, h_ref, mean_ref, max_ref, *, g):
    b = batch_ref[...]
    h = h_ref[...]

    def body(i, carry):
        m = b == i
        cnt = jnp.sum(jnp.where(m, 1.0, 0.0))
        s = jnp.sum(jnp.where(m, h, 0.0), axis=0, keepdims=True)
        mx = jnp.max(jnp.where(m, h, -jnp.inf), axis=0, keepdims=True)
        mean_ref[pl.ds(i, 1), :] = s / jnp.maximum(cnt, 1.0)
        max_ref[pl.ds(i, 1), :] = mx
        return carry

    jax.lax.fori_loop(0, g, body, 0)


def _pool(batch, h, g):
    n, f = h.shape
    nf = f // _FC
    return pl.pallas_call(
        functools.partial(_pool_kernel, g=g),
        grid=(nf,),
        in_specs=[
            pl.BlockSpec((n, 1), lambda j: (0, 0)),
            pl.BlockSpec((n, _FC), lambda j: (0, j)),
        ],
        out_specs=[
            pl.BlockSpec((g, _FC), lambda j: (0, j)),
            pl.BlockSpec((g, _FC), lambda j: (0, j)),
        ],
        out_shape=[
            jax.ShapeDtypeStruct((g, f), jnp.float32),
            jax.ShapeDtypeStruct((g, f), jnp.float32),
        ],
    )(batch, h)


def _head_kernel(z_ref, w1_ref, b1_ref, g_ref, bb_ref, w2_ref, b2_ref, o_ref):
    z = jnp.dot(z_ref[...], w1_ref[...],
                preferred_element_type=jnp.float32) + b1_ref[...]
    mu = jnp.mean(z, axis=0, keepdims=True)
    var = jnp.mean((z - mu) ** 2, axis=0, keepdims=True)
    z = (z - mu) * jax.lax.rsqrt(var + 1e-5) * g_ref[...] + bb_ref[...]
    z = jnp.maximum(z, 0.0)
    logits = jnp.dot(z, w2_ref[...],
                     preferred_element_type=jnp.float32) + b2_ref[...]
    col = jax.lax.broadcasted_iota(jnp.int32, logits.shape, 1)
    logits = jnp.where(col < 2, logits, -jnp.inf)
    mx = jnp.max(logits, axis=1, keepdims=True)
    ex = jnp.exp(logits - mx)
    o_ref[...] = ex / jnp.sum(ex, axis=1, keepdims=True)


def _head(z, w1, b1, bn_g, bn_b, w2, b2):
    g, kdim = z.shape
    return pl.pallas_call(
        _head_kernel,
        out_shape=jax.ShapeDtypeStruct((g, 128), jnp.float32),
    )(z, w1, b1.reshape(1, -1), bn_g.reshape(1, -1), bn_b.reshape(1, -1),
      w2, b2.reshape(1, -1))


def kernel(x, edge_index, batch, lin1_W, lin1_b, lin2_W, lin2_b,
           c1_W, c1_b, c2_W, c2_b, c3_W, c3_b,
           fc1_W, fc1_b, bn_g, bn_b, fc2_W, fc2_b):
    n = x.shape[0]
    g = 64
    h_dim = lin1_W.shape[1]
    ha = h_dim + _AA
    f1p = 3 * _FC
    f2p = 5 * _FC
    f3p = 9 * _FC

    loop = jnp.arange(n, dtype=edge_index.dtype)
    src = jnp.concatenate([edge_index[0], loop]).reshape(-1, 1)
    dst = jnp.concatenate([edge_index[1], loop]).reshape(-1, 1)

    dinv = _deg(dst, n)

    x1 = _mm(x[:, _AA:], lin1_W, lin1_b, relu=True)
    x2 = _mm(_pad_cols(x[:, :_AA], 128), _pad2(lin2_W, 128, 128),
             jnp.pad(lin2_b, (0, 128 - _AA)), relu=True)

    h = jnp.concatenate(
        [x2[:, :_AA], x1, jnp.zeros((n, f1p - ha), jnp.float32)], axis=1)

    h = _gcn_conv(h, src, dst, dinv,
                  _pad2(c1_W, f1p, f1p), jnp.pad(c1_b, (0, f1p - ha)))
    h = _gcn_conv(h, src, dst, dinv,
                  _pad2(c2_W, f1p, f2p), jnp.pad(c2_b, (0, f2p - 2 * ha)))
    h = _gcn_conv(h, src, dst, dinv,
                  _pad2(c3_W, f2p, f3p), jnp.pad(c3_b, (0, f3p - 4 * ha)))

    gmean, gmax = _pool(batch.reshape(-1, 1), h, g)
    z = jnp.concatenate([gmean[:, :4 * ha], gmax[:, :4 * ha]], axis=1)
    zk = z.shape[1]
    zkp = ((zk + 127) // 128) * 128
    out = _head(_pad_cols(z, zkp), _pad2(fc1_W, zkp, 1024), fc1_b,
                bn_g, bn_b, _pad2(fc2_W, 1024, 128),
                jnp.pad(fc2_b, (0, 126)))
    return out[:, :2]
